# baseline (speedup 1.0000x reference)
import functools

import jax
import jax.numpy as jnp
from jax import lax
from jax.experimental import pallas as pl
from jax.experimental.pallas import tpu as pltpu
from jax.experimental.pallas import tpu_sc as plsc

N = 16384
TABLE = 1001
LANES = 16
NUM_CORES = 1
NUM_SUBCORES = 16
NUM_WORKERS = NUM_CORES * NUM_SUBCORES
CHUNK = N // NUM_WORKERS

_MAGIC = 8388608.0

_mesh = plsc.VectorSubcoreMesh(
    core_axis_name="c", subcore_axis_name="s", num_cores=NUM_CORES
)


HALF = CHUNK // 2


@functools.partial(
    pl.kernel,
    mesh=_mesh,
    out_type=jax.ShapeDtypeStruct((N,), jnp.float32),
    scratch_types=[
        pltpu.VMEM((TABLE,), jnp.float32),
        pltpu.VMEM((CHUNK,), jnp.float32),
        pltpu.VMEM((CHUNK,), jnp.float32),
        pltpu.SemaphoreType.DMA,
        pltpu.SemaphoreType.DMA,
        pltpu.SemaphoreType.DMA,
        pltpu.SemaphoreType.DMA,
    ],
    compiler_params=pltpu.CompilerParams(needs_layout_passes=False),
)
def _gamma_lookup(t_hbm, gamma_hbm, out_hbm, tab_v, t_v, o_v, sem_tab, sem_t0, sem_t1, sem_o):
    wid = lax.axis_index("s") * NUM_CORES + lax.axis_index("c")
    base = wid * CHUNK
    cp_tab = pltpu.async_copy(gamma_hbm, tab_v, sem_tab)
    cp_t0 = pltpu.async_copy(t_hbm.at[pl.ds(base, HALF)], t_v.at[pl.ds(0, HALF)], sem_t0)
    cp_t1 = pltpu.async_copy(
        t_hbm.at[pl.ds(base + HALF, HALF)], t_v.at[pl.ds(HALF, HALF)], sem_t1
    )
    cp_tab.wait()
    cp_t0.wait()
    for j in range(HALF // LANES):
        tv = t_v[pl.ds(j * LANES, LANES)]
        idx = ((tv * 1000.0 + _MAGIC) - _MAGIC).astype(jnp.int32)
        o_v[pl.ds(j * LANES, LANES)] = plsc.load_gather(tab_v, [idx])
    cp_o0 = pltpu.async_copy(
        o_v.at[pl.ds(0, HALF)], out_hbm.at[pl.ds(base, HALF)], sem_o
    )
    cp_t1.wait()
    for j in range(HALF // LANES, CHUNK // LANES):
        tv = t_v[pl.ds(j * LANES, LANES)]
        idx = ((tv * 1000.0 + _MAGIC) - _MAGIC).astype(jnp.int32)
        o_v[pl.ds(j * LANES, LANES)] = plsc.load_gather(tab_v, [idx])
    cp_o1 = pltpu.async_copy(
        o_v.at[pl.ds(HALF, HALF)], out_hbm.at[pl.ds(base + HALF, HALF)], sem_o
    )
    cp_o0.wait()
    cp_o1.wait()


def kernel(t, gamma):
    out = _gamma_lookup(t.reshape(N), gamma)
    return out.reshape(N, 1)
